# baseline (speedup 1.0000x reference)
import functools

import jax
import jax.numpy as jnp
from jax import lax
from jax.experimental import pallas as pl
from jax.experimental.pallas import tpu as pltpu
from jax.experimental.pallas import tpu_sc as plsc

ROWS = 1024
COLS = 100000
K = 64
N = ROWS * K

_info = plsc.get_sparse_core_info()
NC = _info.num_cores
NS = _info.num_subcores
NW = NC * NS
RPW = ROWS // NW
EPW = N // NW
SPW = RPW // 8

CT = 49
CW = CT * 128
NCK = 16
TMAX = (COLS + 127) // 128 - CT
NCH = SPW * NCK

_GDN = lax.GatherDimensionNumbers(
    offset_dims=(), collapsed_slice_dims=(0,), start_index_map=(0,)
)

_mesh = plsc.VectorSubcoreMesh(core_axis_name="c", subcore_axis_name="s")


@functools.partial(
    pl.kernel,
    mesh=_mesh,
    out_type=jax.ShapeDtypeStruct((N,), jnp.float32),
    scratch_types=[
        pltpu.VMEM((EPW,), jnp.int32),
        pltpu.VMEM((EPW,), jnp.float32),
        pltpu.VMEM((8, CW), jnp.float32),
        pltpu.VMEM((8, CW), jnp.float32),
        pltpu.SemaphoreType.DMA,
        pltpu.SemaphoreType.DMA,
    ],
)
def _gather_kernel(
    x_hbm, idx_hbm, out_hbm, idx_v, out_v, buf_a, buf_b, sem_a, sem_b
):
    wid = lax.axis_index("s") * NC + lax.axis_index("c")
    ebase = wid * EPW
    rowbase = wid * RPW
    pltpu.sync_copy(idx_hbm.at[pl.ds(ebase, EPW)], idx_v)

    zeros16 = jnp.zeros((16,), jnp.float32)

    @pl.loop(0, EPW // 16)
    def _init(g):
        out_v[pl.ds(g * 16, 16)] = zeros16

    lanes16 = lax.iota(jnp.int32, 16)

    def fire(n, buf, sem):
        stripe = n >> 4
        t0 = jnp.minimum((n & 15) * CT, TMAX)
        row8 = pl.multiple_of(rowbase + stripe * 8, 8)
        c0 = pl.multiple_of(t0 * 128, 128)
        pltpu.async_copy(
            x_hbm.at[pl.ds(row8, 8), pl.ds(c0, CW)], buf, sem
        )

    def drain(buf, sem):
        pltpu.make_async_copy(
            x_hbm.at[pl.ds(0, 8), pl.ds(0, CW)], buf, sem
        ).wait()

    def extract(n, buf):
        stripe = n >> 4
        col0 = jnp.minimum((n & 15) * CT, TMAX) * 128
        gbase = stripe * 32

        @pl.loop(0, 32)
        def _group(gg):
            subl = (gg >> 2) & 7
            g = gbase + gg
            jv = idx_v[pl.ds(g * 16, 16)]
            acc = out_v[pl.ds(g * 16, 16)]
            for t in range(16):
                s = jv[t]
                loc = s - col0
                l2 = jnp.clip(loc, 0, CW - 1)
                v2 = buf[subl, pl.ds(pl.multiple_of(l2 & -16, 16), 16)]
                w = lax.gather(
                    v2,
                    jnp.full((16,), l2 & 15, jnp.int32)[:, None],
                    _GDN,
                    (1,),
                    mode=lax.GatherScatterMode.PROMISE_IN_BOUNDS,
                )
                tsel = jnp.where((loc >= 0) & (loc < CW), t, 16)
                acc = jnp.where(lanes16 == tsel, w, acc)
            out_v[pl.ds(g * 16, 16)] = acc

    fire(0, buf_a, sem_a)
    fire(1, buf_b, sem_b)

    @pl.loop(0, NCH // 2 - 1)
    def _pipeline(k):
        n0 = 2 * k
        drain(buf_a, sem_a)
        extract(n0, buf_a)
        fire(n0 + 2, buf_a, sem_a)
        drain(buf_b, sem_b)
        extract(n0 + 1, buf_b)
        fire(n0 + 3, buf_b, sem_b)

    drain(buf_a, sem_a)
    extract(NCH - 2, buf_a)
    drain(buf_b, sem_b)
    extract(NCH - 1, buf_b)

    pltpu.sync_copy(out_v, out_hbm.at[pl.ds(ebase, EPW)])


def kernel(x, index):
    out = _gather_kernel(x, index.reshape(N))
    return out.reshape(ROWS, K)
